# Initial kernel scaffold; baseline (speedup 1.0000x reference)
import jax
import jax.numpy as jnp
from jax.experimental import pallas as pl


def kernel(features_1, features_2, hb, edge_index_1, edge_index_2, c1_W1, c1_b1, c1_W2, c1_b2, c1_g, c1_bt, c2_W1, c2_b1, c2_W2, c2_b2, c2_g, c2_bt, c3_W1, c3_b1, c3_W2, c3_b2, c3_g, c3_bt, eps, fc1_W, fc2_W, fc3_W, fc3_b, att_W, tn_W, tn_Wb, tn_b, f1_W, f1_b, f2_W, f2_b, f3_W, f3_b, sc_W, sc_b):
    raise NotImplementedError("write your pallas kernel here")


# speedup vs baseline: 6.7654x; 6.7654x over previous
import jax
import jax.numpy as jnp
from jax.experimental import pallas as pl

N = 128
E = 1024


def _bn(h, g, bt):
    m = jnp.mean(h, axis=0, keepdims=True)
    v = jnp.mean((h - m) ** 2, axis=0, keepdims=True)
    return (h - m) * jax.lax.rsqrt(v + 1e-5) * g + bt


def _gin(x, A, eps, W1, b1, W2, b2, g, bt):
    z = (1.0 + eps) * x + jnp.dot(A, x, preferred_element_type=jnp.float32)
    h = jax.nn.relu(jnp.dot(z, W1, preferred_element_type=jnp.float32) + b1)
    h = jnp.dot(h, W2, preferred_element_type=jnp.float32) + b2
    return _bn(h, g, bt)


def _conv_pass(x, A, eps, p):
    h = jax.nn.relu(_gin(x, A, eps[0, 0], p['c1_W1'], p['c1_b1'], p['c1_W2'],
                         p['c1_b2'], p['c1_g'], p['c1_bt']))
    h = jax.nn.relu(_gin(h, A, eps[0, 1], p['c2_W1'], p['c2_b1'], p['c2_W2'],
                         p['c2_b2'], p['c2_g'], p['c2_bt']))
    return _gin(h, A, eps[0, 2], p['c3_W1'], p['c3_b1'], p['c3_W2'],
                p['c3_b2'], p['c3_g'], p['c3_bt'])


def _adjacency(ei_ref):
    src = ei_ref[0:1, :]
    dst = ei_ref[1:2, :]
    iota = jax.lax.broadcasted_iota(jnp.int32, (N, E), 0)
    S = (iota == src).astype(jnp.float32)
    D = (iota == dst).astype(jnp.float32)
    return jax.lax.dot_general(D, S, (((1,), (1,)), ((), ())),
                               preferred_element_type=jnp.float32)


def _att_pool(x, att_W):
    xa = jnp.dot(x, att_W, preferred_element_type=jnp.float32)
    gc = jnp.tanh(jnp.mean(xa, axis=0, keepdims=True))
    s = jax.nn.sigmoid(
        jax.lax.dot_general(x, gc, (((1,), (1,)), ((), ())),
                            preferred_element_type=jnp.float32))
    return jnp.dot(jnp.transpose(s), x,
                   preferred_element_type=jnp.float32)


def _fused_kernel(f1_ref, f2_ref, hb_ref, ei1_ref, ei2_ref,
                  c1_W1, c1_b1, c1_W2, c1_b2, c1_g, c1_bt,
                  c2_W1, c2_b1, c2_W2, c2_b2, c2_g, c2_bt,
                  c3_W1, c3_b1, c3_W2, c3_b2, c3_g, c3_bt,
                  eps_ref, fc1_W, fc2_W, fc3_W, fc3_b, att_W,
                  tn_W, tn_WbT, tn_b, f1_W, f1_b, f2_W, f2_b, f3_W, f3_b,
                  sc_W, sc_b, out_pre, out_score):
    p = {
        'c1_W1': c1_W1[...], 'c1_b1': c1_b1[...], 'c1_W2': c1_W2[...],
        'c1_b2': c1_b2[...], 'c1_g': c1_g[...], 'c1_bt': c1_bt[...],
        'c2_W1': c2_W1[...], 'c2_b1': c2_b1[...], 'c2_W2': c2_W2[...],
        'c2_b2': c2_b2[...], 'c2_g': c2_g[...], 'c2_bt': c2_bt[...],
        'c3_W1': c3_W1[...], 'c3_b1': c3_b1[...], 'c3_W2': c3_W2[...],
        'c3_b2': c3_b2[...], 'c3_g': c3_g[...], 'c3_bt': c3_bt[...],
    }
    eps = eps_ref[...]

    A1 = _adjacency(ei1_ref)
    A2 = _adjacency(ei2_ref)
    h1 = _conv_pass(f1_ref[...], A1, eps, p)
    h2g = _conv_pass(f2_ref[...], A2, eps, p)

    a1 = jnp.dot(h1, fc1_W[...], preferred_element_type=jnp.float32)
    a2 = jnp.dot(h2g, fc1_W[...], preferred_element_type=jnp.float32)
    pair = jax.nn.relu(a1[:, None, :] + a2[None, :, :])
    e2d = pair.reshape(N * N, 64)
    m2 = jax.nn.relu(jnp.dot(e2d, fc2_W[...],
                             preferred_element_type=jnp.float32))
    m3 = m2.reshape(N, N, 32)
    fc3row = fc3_W[...].reshape(1, 1, 32)
    energy = jnp.sum(m3 * fc3row, axis=2) + fc3_b[0, 0]

    emax = jnp.max(energy, axis=1, keepdims=True)
    ex = jnp.exp(energy - emax)
    att = ex / jnp.sum(ex, axis=1, keepdims=True)

    sim = jax.lax.dot_general(h2g, h1, (((1,), (1,)), ((), ())),
                              preferred_element_type=jnp.float32)
    cost = jnp.sum(att * sim)

    p1 = _att_pool(h1, att_W[...])
    p2 = _att_pool(h2g, att_W[...])

    e1c = jnp.transpose(p1).reshape(32, 1, 1)
    e2c = jnp.transpose(p2)
    S1 = jnp.sum(tn_W[...] * e1c, axis=0)
    sc16 = jnp.sum(S1 * e2c, axis=0, keepdims=True)
    comb = jnp.concatenate([p1, p2], axis=1)
    scores = jax.nn.relu(sc16 + jnp.dot(comb, tn_WbT[...],
                                        preferred_element_type=jnp.float32)
                         + tn_b[...])
    scores = jax.nn.relu(jnp.dot(scores, f1_W[...],
                                 preferred_element_type=jnp.float32) + f1_b[...])
    scores = jax.nn.relu(jnp.dot(scores, f2_W[...],
                                 preferred_element_type=jnp.float32) + f2_b[...])
    scores = jax.nn.relu(jnp.dot(scores, f3_W[...],
                                 preferred_element_type=jnp.float32) + f3_b[...])
    bias = jnp.dot(scores, sc_W[...],
                   preferred_element_type=jnp.float32) + sc_b[...]

    score = jax.nn.sigmoid(cost + bias)
    out_score[...] = score
    out_pre[...] = score * hb_ref[...]


def kernel(features_1, features_2, hb, edge_index_1, edge_index_2,
           c1_W1, c1_b1, c1_W2, c1_b2, c1_g, c1_bt,
           c2_W1, c2_b1, c2_W2, c2_b2, c2_g, c2_bt,
           c3_W1, c3_b1, c3_W2, c3_b2, c3_g, c3_bt,
           eps, fc1_W, fc2_W, fc3_W, fc3_b, att_W,
           tn_W, tn_Wb, tn_b,
           f1_W, f1_b, f2_W, f2_b, f3_W, f3_b,
           sc_W, sc_b):
    r = lambda v: v.reshape(1, -1)
    args = (
        features_1, features_2, hb.reshape(1, 1),
        edge_index_1.astype(jnp.int32), edge_index_2.astype(jnp.int32),
        c1_W1, r(c1_b1), c1_W2, r(c1_b2), r(c1_g), r(c1_bt),
        c2_W1, r(c2_b1), c2_W2, r(c2_b2), r(c2_g), r(c2_bt),
        c3_W1, r(c3_b1), c3_W2, r(c3_b2), r(c3_g), r(c3_bt),
        r(eps), fc1_W, fc2_W, fc3_W.reshape(1, 32), r(fc3_b), att_W,
        tn_W, tn_Wb.T, tn_b.reshape(1, 16),
        f1_W, r(f1_b), f2_W, r(f2_b), f3_W, r(f3_b),
        sc_W, r(sc_b),
    )
    out_pre, out_score = pl.pallas_call(
        _fused_kernel,
        out_shape=(jax.ShapeDtypeStruct((1, 1), jnp.float32),
                   jax.ShapeDtypeStruct((1, 1), jnp.float32)),
    )(*args)
    return (out_pre.reshape(-1), out_score.reshape(-1))
